# baseline (speedup 1.0000x reference)
import functools

import jax
import jax.numpy as jnp
from jax import lax
from jax.experimental import pallas as pl
from jax.experimental.pallas import tpu as pltpu
from jax.experimental.pallas import tpu_sc as plsc

B = 4096
L = 200
D = 64
SCALE = float(D) ** 0.5
PAD = 0
V = 1000001

_info = plsc.get_sparse_core_info()
NC = _info.num_cores
NS = _info.num_subcores
NW = NC * NS
BBLK = B // NW
NBUF = 3
CL = 2
LANES = 16
LT, LW = L // 8, 8
DT, DW = D // 8, 8

TBLK = 512
HALF = ((V // 2 + TBLK) // TBLK) * TBLK
PACKROWS = 2 * HALF


def _pack_body(lo_ref, hi_ref, out_ref):
    out_ref[...] = jnp.concatenate([lo_ref[...].T, hi_ref[...].T], axis=1)


@jax.jit
def _tc_pack_table(item_emb):
    item_t = item_emb.T
    nk = HALF // TBLK
    packed = pl.pallas_call(
        _pack_body,
        grid=(nk,),
        in_specs=[
            pl.BlockSpec((D, TBLK), lambda k: (0, k)),
            pl.BlockSpec((D, TBLK), lambda k, _nk=nk: (0, _nk + k)),
        ],
        out_specs=pl.BlockSpec((TBLK, 2 * D), lambda k: (k, 0)),
        out_shape=jax.ShapeDtypeStruct((HALF, 2 * D), jnp.float32),
    )(item_t, item_t)
    return packed.reshape(PACKROWS, D)


def _sc_embed_body(idx4_hbm, item_hbm, pos_hbm, out_hbm,
                   pos_v, idx0, idx1, idx2, g0, g1, g2, ob0, ob1, ob2,
                   isem0, isem1, isem2, gsem0, gsem1, gsem2,
                   wsem0, wsem1, wsem2):
    idxb = [idx0, idx1, idx2]
    gbuf = [g0, g1, g2]
    obuf = [ob0, ob1, ob2]
    isem = [isem0, isem1, isem2]
    gsem = [gsem0, gsem1, gsem2]
    wsem = [wsem0, wsem1, wsem2]

    w = lax.axis_index("s") * NC + lax.axis_index("c")

    pltpu.sync_copy(pos_hbm, pos_v)

    def start_idx(s, b):
        l0 = s * CL
        pltpu.async_copy(idx4_hbm.at[l0 // LW, w, pl.ds(l0 % LW, CL)],
                         idxb[b], isem[b])

    def start_gather(b):
        pltpu.make_async_copy(idx4_hbm.at[0, 0, pl.ds(0, CL)], idxb[b],
                              isem[b]).wait()
        for c in range(CL):
            for j in range(BBLK // LANES):
                sl = pl.ds(j * LANES, LANES)
                r = idxb[b][c, sl]
                idxb[b][c, sl] = r * 2 - jnp.where(
                    r < HALF, jnp.int32(0), jnp.int32(2 * HALF - 1))
        for c in range(CL):
            pltpu.async_copy(item_hbm.at[idxb[b].at[c]], gbuf[b].at[c],
                             gsem[b])

    def drain_gather(b):
        for c in range(CL):
            pltpu.make_async_copy(item_hbm.at[pl.ds(0, BBLK)],
                                  gbuf[b].at[c], gsem[b]).wait()

    def start_write(s, b):
        l0 = s * CL
        for c in range(CL):
            for dt in range(DT):
                pltpu.async_copy(obuf[b].at[c, dt],
                                 out_hbm.at[l0 + c, dt, w], wsem[b])

    def drain_write(b):
        for c in range(CL):
            for dt in range(DT):
                pltpu.make_async_copy(obuf[b].at[c, dt],
                                      out_hbm.at[0, dt, 0], wsem[b]).wait()

    def compute(s, b):
        row16 = lax.iota(jnp.int32, LANES)
        rows_j = [row16 + (j * LANES) for j in range(BBLK // LANES)]
        l0 = s * CL
        for c in range(CL):
            lvec = jnp.full((LANES,), l0 + c, jnp.int32)
            cvec = jnp.full((LANES,), c, jnp.int32)

            def d_body(d, c=c, lvec=lvec, cvec=cvec):
                dt = d // DW
                dw = d % DW
                cols = jnp.full((LANES,), d, jnp.int32)
                pvec = plsc.load_gather(pos_v, [lvec, cols])
                for j in range(BBLK // LANES):
                    vals = plsc.load_gather(gbuf[b], [cvec, rows_j[j], cols])
                    obuf[b][c, dt, dw, pl.ds(j * LANES, LANES)] = (
                        vals * SCALE + pvec)

            plsc.parallel_loop(0, D, 1, unroll=2)(d_body)

    NS_SLOTS = L // CL

    for p in range(NBUF):
        start_idx(p, p)
    for p in range(NBUF - 1):
        start_gather(p)

    def step(s, b, o):
        @pl.when(s + NBUF - 1 < NS_SLOTS)
        def _():
            start_gather((b + NBUF - 1) % NBUF)

        drain_gather(b)

        @pl.when(o > 0)
        def _():
            drain_write(b)

        compute(s, b)
        start_write(s, b)

        @pl.when(s + NBUF < NS_SLOTS)
        def _():
            start_idx(s + NBUF, b)

    def outer(o, carry):
        for b in range(NBUF):
            step(o * NBUF + b, b, o)
        return carry

    lax.fori_loop(0, NS_SLOTS // NBUF, outer, 0)

    REM = NS_SLOTS % NBUF
    for t in range(REM):
        s = NS_SLOTS - REM + t
        b = s % NBUF
        drain_gather(b)
        drain_write(b)
        compute(s, b)
        start_write(s, b)

    for b in range(NBUF):
        drain_write(b)


@jax.jit
def _sc_embed(log_seqs, item_emb, pos_emb):
    idx4 = log_seqs.reshape(B // 128, 128, LT, LW).transpose(2, 0, 3, 1)
    item_packed = _tc_pack_table(item_emb)
    kern = functools.partial(
        pl.kernel,
        out_type=jax.ShapeDtypeStruct((L, DT, NW, DW, 128), jnp.float32),
        mesh=plsc.VectorSubcoreMesh(core_axis_name="c", subcore_axis_name="s"),
        compiler_params=pltpu.CompilerParams(use_tc_tiling_on_sc=False,
                                             needs_layout_passes=False),
        scratch_types=[
            pltpu.VMEM((L, D), jnp.float32),
            pltpu.VMEM((CL, 128), jnp.int32),
            pltpu.VMEM((CL, 128), jnp.int32),
            pltpu.VMEM((CL, 128), jnp.int32),
            pltpu.VMEM((CL, BBLK, D), jnp.float32),
            pltpu.VMEM((CL, BBLK, D), jnp.float32),
            pltpu.VMEM((CL, BBLK, D), jnp.float32),
            pltpu.VMEM((CL, DT, DW, 128), jnp.float32),
            pltpu.VMEM((CL, DT, DW, 128), jnp.float32),
            pltpu.VMEM((CL, DT, DW, 128), jnp.float32),
            pltpu.SemaphoreType.DMA,
            pltpu.SemaphoreType.DMA,
            pltpu.SemaphoreType.DMA,
            pltpu.SemaphoreType.DMA,
            pltpu.SemaphoreType.DMA,
            pltpu.SemaphoreType.DMA,
            pltpu.SemaphoreType.DMA,
            pltpu.SemaphoreType.DMA,
            pltpu.SemaphoreType.DMA,
        ],
    )(_sc_embed_body)
    out5 = kern(idx4, item_packed, pos_emb)
    return out5.transpose(2, 4, 0, 1, 3).reshape(B, L, D)


def _mask_body(seq_ref, mask_ref):
    mask_ref[...] = seq_ref[...] == PAD


@jax.jit
def _tc_mask(log_seqs):
    return pl.pallas_call(
        _mask_body,
        out_shape=jax.ShapeDtypeStruct((B, L), jnp.bool_),
    )(log_seqs)


def kernel(log_seqs, item_emb, pos_emb):
    log_seqs = log_seqs.astype(jnp.int32)
    seqs = _sc_embed(log_seqs, item_emb, pos_emb)
    mask = _tc_mask(log_seqs)
    return seqs, mask


# speedup vs baseline: 2.5162x; 2.5162x over previous
import functools

import jax
import jax.numpy as jnp
from jax import lax
from jax.experimental import pallas as pl
from jax.experimental.pallas import tpu as pltpu
from jax.experimental.pallas import tpu_sc as plsc

B = 4096
L = 200
D = 64
SCALE = float(D) ** 0.5
PAD = 0
V = 1000001

_info = plsc.get_sparse_core_info()
NC = _info.num_cores
NS = _info.num_subcores
NW = NC * NS
BBLK = B // NW
NBUF = 3
CL = 2
LANES = 16
LT, LW = L // 8, 8
DT, DW = D // 8, 8

TBLK = 2048
HALF = ((V // 2 + TBLK) // TBLK) * TBLK
GPITCH = 65
PACKROWS = 2 * HALF


def _pack_body(lo_ref, hi_ref, out_ref):
    out_ref[...] = jnp.concatenate([lo_ref[...].T, hi_ref[...].T], axis=1)


@jax.jit
def _tc_pack_table(item_emb):
    item_t = item_emb.T
    nk = HALF // TBLK
    last_blk = (V - 1) // TBLK
    packed = pl.pallas_call(
        _pack_body,
        grid=(nk,),
        in_specs=[
            pl.BlockSpec((D, TBLK), lambda k: (0, k)),
            pl.BlockSpec((D, TBLK),
                         lambda k, _nk=nk, _lb=last_blk:
                         (0, jnp.minimum(_nk + k, _lb))),
        ],
        out_specs=pl.BlockSpec((TBLK, 2 * D), lambda k: (k, 0)),
        out_shape=jax.ShapeDtypeStruct((HALF, 2 * D), jnp.float32),
    )(item_t, item_t)
    return packed.reshape(PACKROWS, D)


def _sc_embed_body(idx4_hbm, item_hbm, pos_hbm, out_hbm,
                   pos_v, g2s, idx0, idx1, idx2, g0, g1, g2, ob0, ob1, ob2,
                   isem0, isem1, isem2, gsem0, gsem1, gsem2,
                   wsem0, wsem1, wsem2):
    idxb = [idx0, idx1, idx2]
    gbuf = [g0, g1, g2]
    obuf = [ob0, ob1, ob2]
    isem = [isem0, isem1, isem2]
    gsem = [gsem0, gsem1, gsem2]
    wsem = [wsem0, wsem1, wsem2]

    w = lax.axis_index("s") * NC + lax.axis_index("c")

    pltpu.sync_copy(pos_hbm, pos_v)

    def start_idx(s, b):
        l0 = s * CL
        pltpu.async_copy(idx4_hbm.at[l0 // LW, w, pl.ds(l0 % LW, CL)],
                         idxb[b], isem[b])

    def start_gather(b):
        pltpu.make_async_copy(idx4_hbm.at[0, 0, pl.ds(0, CL)], idxb[b],
                              isem[b]).wait()
        for c in range(CL):
            for j in range(BBLK // LANES):
                sl = pl.ds(j * LANES, LANES)
                r = idxb[b][c, sl]
                idxb[b][c, sl] = r * 2 - jnp.where(
                    r < HALF, jnp.int32(0), jnp.int32(2 * HALF - 1))
        for c in range(CL):
            pltpu.async_copy(item_hbm.at[idxb[b].at[c]], gbuf[b].at[c],
                             gsem[b])

    def drain_gather(b):
        for c in range(CL):
            pltpu.make_async_copy(item_hbm.at[pl.ds(0, BBLK)],
                                  gbuf[b].at[c], gsem[b]).wait()

    def start_write(s, b):
        l0 = s * CL
        for c in range(CL):
            for dt in range(DT):
                pltpu.async_copy(obuf[b].at[c, dt],
                                 out_hbm.at[l0 + c, dt, w], wsem[b])

    def drain_write(b):
        for c in range(CL):
            for dt in range(DT):
                pltpu.make_async_copy(obuf[b].at[c, dt],
                                      out_hbm.at[0, dt, 0], wsem[b]).wait()

    def compute(s, b):
        row16 = lax.iota(jnp.int32, LANES)
        rows_j = [row16 + j * LANES for j in range(BBLK // LANES)]
        l0 = s * CL
        for c in range(CL):
            pv = [pos_v[l0 + c, pl.ds(k * LANES, LANES)]
                  for k in range(D // LANES)]

            def stage_body(bw, c=c, pv=pv):
                for k in range(D // LANES):
                    sl = pl.ds(k * LANES, LANES)
                    g2s[bw, sl] = gbuf[b][c, bw, sl] * SCALE + pv[k]

            plsc.parallel_loop(0, BBLK, 1, unroll=4)(stage_body)

            def d_body(d, c=c):
                dt = d // DW
                dw = d % DW
                cols = jnp.full((LANES,), d, jnp.int32)
                for j in range(BBLK // LANES):
                    vals = plsc.load_gather(g2s, [rows_j[j], cols])
                    obuf[b][c, dt, dw, pl.ds(j * LANES, LANES)] = vals

            plsc.parallel_loop(0, D, 1, unroll=2)(d_body)

    NS_SLOTS = L // CL

    for p in range(NBUF):
        start_idx(p, p)
    for p in range(NBUF - 1):
        start_gather(p)

    def step(s, b, o):
        @pl.when(s + NBUF - 1 < NS_SLOTS)
        def _():
            start_gather((b + NBUF - 1) % NBUF)

        drain_gather(b)

        @pl.when(o > 0)
        def _():
            drain_write(b)

        compute(s, b)
        start_write(s, b)

        @pl.when(s + NBUF < NS_SLOTS)
        def _():
            start_idx(s + NBUF, b)

    def outer(o, carry):
        for b in range(NBUF):
            step(o * NBUF + b, b, o)
        return carry

    lax.fori_loop(0, NS_SLOTS // NBUF, outer, 0)

    REM = NS_SLOTS % NBUF
    for t in range(REM):
        s = NS_SLOTS - REM + t
        b = s % NBUF
        drain_gather(b)
        drain_write(b)
        compute(s, b)
        start_write(s, b)

    for b in range(NBUF):
        drain_write(b)


@jax.jit
def _sc_embed(log_seqs, item_emb, pos_emb):
    idx4 = log_seqs.reshape(B // 128, 128, LT, LW).transpose(2, 0, 3, 1)
    item_packed = _tc_pack_table(item_emb)
    kern = functools.partial(
        pl.kernel,
        out_type=jax.ShapeDtypeStruct((L, DT, NW, DW, 128), jnp.float32),
        mesh=plsc.VectorSubcoreMesh(core_axis_name="c", subcore_axis_name="s"),
        compiler_params=pltpu.CompilerParams(use_tc_tiling_on_sc=False,
                                             needs_layout_passes=False),
        scratch_types=[
            pltpu.VMEM((L, D), jnp.float32),
            pltpu.VMEM((BBLK, GPITCH), jnp.float32),
            pltpu.VMEM((CL, 128), jnp.int32),
            pltpu.VMEM((CL, 128), jnp.int32),
            pltpu.VMEM((CL, 128), jnp.int32),
            pltpu.VMEM((CL, BBLK, D), jnp.float32),
            pltpu.VMEM((CL, BBLK, D), jnp.float32),
            pltpu.VMEM((CL, BBLK, D), jnp.float32),
            pltpu.VMEM((CL, DT, DW, 128), jnp.float32),
            pltpu.VMEM((CL, DT, DW, 128), jnp.float32),
            pltpu.VMEM((CL, DT, DW, 128), jnp.float32),
            pltpu.SemaphoreType.DMA,
            pltpu.SemaphoreType.DMA,
            pltpu.SemaphoreType.DMA,
            pltpu.SemaphoreType.DMA,
            pltpu.SemaphoreType.DMA,
            pltpu.SemaphoreType.DMA,
            pltpu.SemaphoreType.DMA,
            pltpu.SemaphoreType.DMA,
            pltpu.SemaphoreType.DMA,
        ],
    )(_sc_embed_body)
    out5 = kern(idx4, item_packed, pos_emb)
    return out5.transpose(2, 4, 0, 1, 3).reshape(B, L, D)


def _mask_body(seq_ref, mask_ref):
    mask_ref[...] = seq_ref[...] == PAD


@jax.jit
def _tc_mask(log_seqs):
    return pl.pallas_call(
        _mask_body,
        out_shape=jax.ShapeDtypeStruct((B, L), jnp.bool_),
    )(log_seqs)


def kernel(log_seqs, item_emb, pos_emb):
    log_seqs = log_seqs.astype(jnp.int32)
    seqs = _sc_embed(log_seqs, item_emb, pos_emb)
    mask = _tc_mask(log_seqs)
    return seqs, mask


# speedup vs baseline: 2.8551x; 1.1347x over previous
import functools

import jax
import jax.numpy as jnp
from jax import lax
from jax.experimental import pallas as pl
from jax.experimental.pallas import tpu as pltpu
from jax.experimental.pallas import tpu_sc as plsc

B = 4096
L = 200
D = 64
SCALE = float(D) ** 0.5
PAD = 0
V = 1000001

_info = plsc.get_sparse_core_info()
NC = _info.num_cores
NS = _info.num_subcores
NW = NC * NS
BBLK = B // NW
NBUF = 3
CL = 2
LANES = 16
LT, LW = L // 8, 8
DT, DW = D // 8, 8

TBLK = 4096
HALF = ((V // 2 + TBLK) // TBLK) * TBLK
GPITCH = 65
PACKROWS = 2 * HALF


def _pack_body(lo_ref, hi_ref, out_ref):
    r = lax.broadcasted_iota(jnp.int32, (D, D), 0)
    c = lax.broadcasted_iota(jnp.int32, (D, D), 1)
    eye = (r == c).astype(jnp.float32)
    dn = (((0,), (0,)), ((), ()))
    lo_t = lax.dot_general(lo_ref[...], eye, dn,
                           preferred_element_type=jnp.float32)
    hi_t = lax.dot_general(hi_ref[...], eye, dn,
                           preferred_element_type=jnp.float32)
    out_ref[...] = jnp.concatenate([lo_t, hi_t], axis=1)


@jax.jit
def _tc_pack_table(item_emb):
    item_t = item_emb.T
    nk = HALF // TBLK
    last_blk = (V - 1) // TBLK
    packed = pl.pallas_call(
        _pack_body,
        grid=(nk,),
        in_specs=[
            pl.BlockSpec((D, TBLK), lambda k: (0, k)),
            pl.BlockSpec((D, TBLK),
                         lambda k, _nk=nk, _lb=last_blk:
                         (0, jnp.minimum(_nk + k, _lb))),
        ],
        out_specs=pl.BlockSpec((TBLK, 2 * D), lambda k: (k, 0)),
        out_shape=jax.ShapeDtypeStruct((HALF, 2 * D), jnp.float32),
    )(item_t, item_t)
    return packed.reshape(PACKROWS, D)


def _sc_embed_body(idx4_hbm, item_hbm, pos_hbm, out_hbm,
                   pos_v, g2s, idx0, idx1, idx2, g0, g1, g2, ob0, ob1, ob2,
                   isem0, isem1, isem2, gsem0, gsem1, gsem2,
                   wsem0, wsem1, wsem2):
    idxb = [idx0, idx1, idx2]
    gbuf = [g0, g1, g2]
    obuf = [ob0, ob1, ob2]
    isem = [isem0, isem1, isem2]
    gsem = [gsem0, gsem1, gsem2]
    wsem = [wsem0, wsem1, wsem2]

    w = lax.axis_index("s") * NC + lax.axis_index("c")

    pltpu.sync_copy(pos_hbm, pos_v)

    def start_idx(s, b):
        l0 = s * CL
        pltpu.async_copy(idx4_hbm.at[l0 // LW, w, pl.ds(l0 % LW, CL)],
                         idxb[b], isem[b])

    def start_gather(b):
        pltpu.make_async_copy(idx4_hbm.at[0, 0, pl.ds(0, CL)], idxb[b],
                              isem[b]).wait()
        for c in range(CL):
            for j in range(BBLK // LANES):
                sl = pl.ds(j * LANES, LANES)
                r = idxb[b][c, sl]
                idxb[b][c, sl] = r * 2 - jnp.where(
                    r < HALF, jnp.int32(0), jnp.int32(2 * HALF - 1))
        for c in range(CL):
            pltpu.async_copy(item_hbm.at[idxb[b].at[c]], gbuf[b].at[c],
                             gsem[b])

    def drain_gather(b):
        for c in range(CL):
            pltpu.make_async_copy(item_hbm.at[pl.ds(0, BBLK)],
                                  gbuf[b].at[c], gsem[b]).wait()

    def start_write(s, b):
        l0 = s * CL
        for c in range(CL):
            for dt in range(DT):
                pltpu.async_copy(obuf[b].at[c, dt],
                                 out_hbm.at[l0 + c, dt, w], wsem[b])

    def drain_write(b):
        for c in range(CL):
            for dt in range(DT):
                pltpu.make_async_copy(obuf[b].at[c, dt],
                                      out_hbm.at[0, dt, 0], wsem[b]).wait()

    def compute(s, b):
        row16 = lax.iota(jnp.int32, LANES)
        rows_j = [row16 + j * LANES for j in range(BBLK // LANES)]
        l0 = s * CL
        for c in range(CL):
            pv = [pos_v[l0 + c, pl.ds(k * LANES, LANES)]
                  for k in range(D // LANES)]

            def stage_body(bw, c=c, pv=pv):
                for k in range(D // LANES):
                    sl = pl.ds(k * LANES, LANES)
                    g2s[bw, sl] = gbuf[b][c, bw, sl] * SCALE + pv[k]

            plsc.parallel_loop(0, BBLK, 1, unroll=4)(stage_body)

            def d_body(d, c=c):
                dt = d // DW
                dw = d % DW
                cols = jnp.full((LANES,), d, jnp.int32)
                for j in range(BBLK // LANES):
                    vals = plsc.load_gather(g2s, [rows_j[j], cols])
                    obuf[b][c, dt, dw, pl.ds(j * LANES, LANES)] = vals

            plsc.parallel_loop(0, D, 1, unroll=2)(d_body)

    NS_SLOTS = L // CL

    for p in range(NBUF):
        start_idx(p, p)
    for p in range(NBUF - 1):
        start_gather(p)

    def step(s, b, o):
        @pl.when(s + NBUF - 1 < NS_SLOTS)
        def _():
            start_gather((b + NBUF - 1) % NBUF)

        drain_gather(b)

        @pl.when(o > 0)
        def _():
            drain_write(b)

        compute(s, b)
        start_write(s, b)

        @pl.when(s + NBUF < NS_SLOTS)
        def _():
            start_idx(s + NBUF, b)

    def outer(o, carry):
        for b in range(NBUF):
            step(o * NBUF + b, b, o)
        return carry

    lax.fori_loop(0, NS_SLOTS // NBUF, outer, 0)

    REM = NS_SLOTS % NBUF
    for t in range(REM):
        s = NS_SLOTS - REM + t
        b = s % NBUF
        drain_gather(b)
        drain_write(b)
        compute(s, b)
        start_write(s, b)

    for b in range(NBUF):
        drain_write(b)


@jax.jit
def _sc_embed(log_seqs, item_emb, pos_emb):
    idx4 = log_seqs.reshape(B // 128, 128, LT, LW).transpose(2, 0, 3, 1)
    item_packed = _tc_pack_table(item_emb)
    kern = functools.partial(
        pl.kernel,
        out_type=jax.ShapeDtypeStruct((L, DT, NW, DW, 128), jnp.float32),
        mesh=plsc.VectorSubcoreMesh(core_axis_name="c", subcore_axis_name="s"),
        compiler_params=pltpu.CompilerParams(use_tc_tiling_on_sc=False,
                                             needs_layout_passes=False),
        scratch_types=[
            pltpu.VMEM((L, D), jnp.float32),
            pltpu.VMEM((BBLK, GPITCH), jnp.float32),
            pltpu.VMEM((CL, 128), jnp.int32),
            pltpu.VMEM((CL, 128), jnp.int32),
            pltpu.VMEM((CL, 128), jnp.int32),
            pltpu.VMEM((CL, BBLK, D), jnp.float32),
            pltpu.VMEM((CL, BBLK, D), jnp.float32),
            pltpu.VMEM((CL, BBLK, D), jnp.float32),
            pltpu.VMEM((CL, DT, DW, 128), jnp.float32),
            pltpu.VMEM((CL, DT, DW, 128), jnp.float32),
            pltpu.VMEM((CL, DT, DW, 128), jnp.float32),
            pltpu.SemaphoreType.DMA,
            pltpu.SemaphoreType.DMA,
            pltpu.SemaphoreType.DMA,
            pltpu.SemaphoreType.DMA,
            pltpu.SemaphoreType.DMA,
            pltpu.SemaphoreType.DMA,
            pltpu.SemaphoreType.DMA,
            pltpu.SemaphoreType.DMA,
            pltpu.SemaphoreType.DMA,
        ],
    )(_sc_embed_body)
    out5 = kern(idx4, item_packed, pos_emb)
    return out5.transpose(2, 4, 0, 1, 3).reshape(B, L, D)


def _mask_body(seq_ref, mask_ref):
    mask_ref[...] = seq_ref[...] == PAD


@jax.jit
def _tc_mask(log_seqs):
    return pl.pallas_call(
        _mask_body,
        out_shape=jax.ShapeDtypeStruct((B, L), jnp.bool_),
    )(log_seqs)


def kernel(log_seqs, item_emb, pos_emb):
    log_seqs = log_seqs.astype(jnp.int32)
    seqs = _sc_embed(log_seqs, item_emb, pos_emb)
    mask = _tc_mask(log_seqs)
    return seqs, mask
